# baseline (speedup 1.0000x reference)
import functools

import jax
import jax.numpy as jnp
from jax import lax
from jax.experimental import pallas as pl
from jax.experimental.pallas import tpu as pltpu
from jax.experimental.pallas import tpu_sc as plsc

_NUM_BANDS = 64
_EMBED_DIM = 128
_B = 4096
_NC = 2
_NS = 16
_NW = _NC * _NS
_BPW = _B // _NW

_mesh = plsc.VectorSubcoreMesh(core_axis_name="c", subcore_axis_name="s")


@functools.partial(
    pl.kernel,
    mesh=_mesh,
    out_type=jax.ShapeDtypeStruct((_B, _NUM_BANDS, _EMBED_DIM), jnp.float32),
    scratch_types=[
        pltpu.VMEM((1, _NUM_BANDS, _EMBED_DIM), jnp.float32),
        pltpu.SemaphoreType.DMA,
    ],
)
def _broadcast_sc(table_hbm, out_hbm, buf, sem):
    wid = lax.axis_index("s") * _NC + lax.axis_index("c")
    pltpu.sync_copy(table_hbm, buf.at[0])
    copies = []
    for i in range(_BPW):
        copies.append(
            pltpu.async_copy(buf, out_hbm.at[pl.ds(wid + i * _NW, 1)], sem)
        )
    for c in copies:
        c.wait()


def kernel(embedding_weight, batch_size):
    del batch_size
    return _broadcast_sc(embedding_weight)
